# baseline (speedup 1.0000x reference)
import jax
import jax.numpy as jnp
from jax import lax
from jax.experimental import pallas as pl
from jax.experimental.pallas import tpu as pltpu
from jax.experimental.pallas import tpu_sc as plsc

N, M, D = 32768, 2048, 256
K = 5
NC, NS, L = 2, 16, 16
NW = NC * NS
CHUNK = N // NW
VPC = CHUNK // L
BINS_W = M // NW
ROWS_W = (K * M) // NW
PAD = NW * L

_mesh = plsc.VectorSubcoreMesh(
    core_axis_name="c", subcore_axis_name="s", num_cores=NC, num_subcores=NS
)


def _wid():
    return lax.axis_index("s") * NC + lax.axis_index("c")


def _take(v, idx):
    return v.at[idx].get(mode="promise_in_bounds")


def _kern_a(ids_hbm, hist_hbm, la_hbm, ids_v, hist_v, la_v):
    w = _wid()
    base = w * CHUNK
    pltpu.sync_copy(ids_hbm.at[pl.ds(base, CHUNK)], ids_v)
    zero16 = jnp.zeros((L,), jnp.int32)

    def zbody(i, carry):
        hist_v[pl.ds(i * L, L)] = zero16
        return carry

    lax.fori_loop(0, M // L, zbody, 0)

    iota = lax.iota(jnp.int32, L)

    def body(tt, carry):
        t = VPC - 1 - tt
        v = ids_v[pl.ds(t * L, L)]
        later = jnp.zeros((L,), jnp.int32)
        tot = jnp.zeros((L,), jnp.int32)
        for s in range(1, L):
            rvec = _take(v, (iota + s) & (L - 1))
            eq = (rvec == v).astype(jnp.int32)
            tot = tot + eq
            later = later + eq * (iota < L - s).astype(jnp.int32)
        aft = plsc.load_gather(hist_v, [v])
        la_v[pl.ds(t * L, L)] = later + aft
        is_first = (tot - later) == 0
        plsc.addupdate_scatter(hist_v, [v], tot + 1, mask=is_first)
        return carry

    lax.fori_loop(0, VPC, body, 0)
    pltpu.sync_copy(hist_v, hist_hbm.at[pl.ds(w * M, M)])
    pltpu.sync_copy(la_v, la_hbm.at[pl.ds(base, CHUNK)])


def _kern_b1(hist_hbm, after_hbm, len_hbm, col_v, after_v, len_v, sem):
    w = _wid()
    b0 = w * BINS_W
    cps = [
        pltpu.async_copy(
            hist_hbm.at[pl.ds(wp * M + b0, BINS_W)], col_v.at[wp], sem
        )
        for wp in range(NW)
    ]
    for cp in cps:
        cp.wait()
    for q in range(BINS_W // L):
        acc = jnp.zeros((L,), jnp.int32)
        for wp in range(NW - 1, -1, -1):
            after_v[wp, pl.ds(q * L, L)] = acc
            acc = acc + col_v[wp, pl.ds(q * L, L)]
        len_v[pl.ds(q * L, L)] = jnp.minimum(acc, K)
    cps = [
        pltpu.async_copy(
            after_v.at[wp], after_hbm.at[pl.ds(wp * M + b0, BINS_W)], sem
        )
        for wp in range(NW)
    ]
    for cp in cps:
        cp.wait()
    pltpu.sync_copy(len_v, len_hbm.at[pl.ds(b0, BINS_W)])


TBL_S = K * M + NW * 128
TBL_T = M + NW * 128
SRC_SLC = K * M // NS
TS_SLC = M // NS


def _kern_b2(ids_hbm, la_hbm, ts_hbm, after_hbm, len_hbm, src2_out, uts2_out,
             ids_v, la_v, ts_v, after_v, len_v, didx_v, dts_v, vals_v,
             tsv_v, zsrc_v, zts_v, spm_src, spm_ts, semi, sem1, sem2):
    w = _wid()
    c = lax.axis_index("c")
    s = lax.axis_index("s")
    base = w * CHUNK
    zero16 = jnp.zeros((L,), jnp.int32)
    zero16f = jnp.zeros((L,), jnp.float32)

    def z1(i, carry):
        zsrc_v[pl.ds(i * L, L)] = zero16
        return carry

    lax.fori_loop(0, SRC_SLC // L, z1, 0)

    def z2(i, carry):
        zts_v[pl.ds(i * L, L)] = zero16f
        return carry

    lax.fori_loop(0, TS_SLC // L, z2, 0)
    pltpu.sync_copy(zsrc_v, spm_src.at[pl.ds(s * SRC_SLC, SRC_SLC)])
    pltpu.sync_copy(zts_v, spm_ts.at[pl.ds(s * TS_SLC, TS_SLC)])
    cps = [
        pltpu.async_copy(ids_hbm.at[pl.ds(base, CHUNK)], ids_v, semi),
        pltpu.async_copy(la_hbm.at[pl.ds(base, CHUNK)], la_v, semi),
        pltpu.async_copy(ts_hbm.at[pl.ds(base, CHUNK)], ts_v, semi),
        pltpu.async_copy(after_hbm.at[pl.ds(w * M, M)], after_v, semi),
        pltpu.async_copy(len_hbm, len_v, semi),
    ]
    for cp in cps:
        cp.wait()
    iota = lax.iota(jnp.int32, L)

    def body(j, carry):
        sl = pl.ds(j * L, L)
        m = ids_v[sl]
        la = la_v[sl]
        aft = plsc.load_gather(after_v, [m])
        pfe = la + aft
        ln = plsc.load_gather(len_v, [m])
        keep = pfe < K
        slot = ln - 1 - pfe
        dump = w * 128 + (j % 8) * L + iota
        d = jnp.where(keep, slot * M + m, K * M + dump)
        dts = jnp.where(pfe == 0, m, M + dump)
        row = j // 8
        col = (j % 8) * L
        didx_v[row, pl.ds(col, L)] = d
        dts_v[row, pl.ds(col, L)] = dts
        vals_v[row, pl.ds(col, L)] = base + j * L + iota + 1
        tsv_v[row, pl.ds(col, L)] = ts_v[sl]
        return carry

    lax.fori_loop(0, VPC, body, 0)
    plsc.subcore_barrier()
    cps = []
    for i in range(CHUNK // 128):
        cps.append(
            pltpu.async_copy(vals_v.at[i], spm_src.at[didx_v.at[i]], sem1, add=True)
        )
        cps.append(
            pltpu.async_copy(tsv_v.at[i], spm_ts.at[dts_v.at[i]], sem2, add=True)
        )
    for cp in cps:
        cp.wait()
    plsc.subcore_barrier()
    pltpu.sync_copy(
        spm_src.at[pl.ds(s * SRC_SLC, SRC_SLC)],
        src2_out.at[pl.ds(c * (K * M) + s * SRC_SLC, SRC_SLC)],
    )
    pltpu.sync_copy(
        spm_ts.at[pl.ds(s * TS_SLC, TS_SLC)],
        uts2_out.at[pl.ds(c * M + s * TS_SLC, TS_SLC)],
    )


def _kern_c(msg_hbm, src2_hbm, uts2_hbm, len_hbm, out_hbm, uts_hbm,
            sflat_v, sflatb_v, sidx_v, len_v, maskf_v, utsm_v, rows_v,
            semi, semg, semw):
    w = _wid()
    base = w * ROWS_W
    cps_in = [
        pltpu.async_copy(src2_hbm.at[pl.ds(base, ROWS_W)], sflat_v, semi),
        pltpu.async_copy(src2_hbm.at[pl.ds(K * M + base, ROWS_W)], sflatb_v, semi),
        pltpu.async_copy(len_hbm, len_v, semi),
        pltpu.async_copy(uts2_hbm.at[pl.ds(w * BINS_W, BINS_W)], utsm_v.at[0], semi),
        pltpu.async_copy(
            uts2_hbm.at[pl.ds(M + w * BINS_W, BINS_W)], utsm_v.at[1], semi
        ),
    ]
    for cp in cps_in:
        cp.wait()
    iota = lax.iota(jnp.int32, L)
    nmax = jnp.full((L,), N - 1, jnp.int32)
    zero = jnp.zeros((L,), jnp.int32)

    def cbody(q, carry):
        v = sflat_v[pl.ds(q * L, L)] + sflatb_v[pl.ds(q * L, L)] - 1
        v = jnp.minimum(jnp.maximum(v, zero), nmax)
        row = q // 5
        col = (q % 5) * L
        sidx_v[row, pl.ds(col, L)] = v
        d = base + q * L + iota
        m = d & (M - 1)
        s = jax.lax.shift_right_logical(d, 11)
        ln = plsc.load_gather(len_v, [m])
        maskf_v[pl.ds(q * L, L)] = (s < ln).astype(jnp.float32)
        return carry

    NWIN = ROWS_W // 80
    gcps = []
    for win in range(NWIN):
        lax.fori_loop(win * 5, win * 5 + 5, cbody, 0)
        gcps.append(
            pltpu.async_copy(
                msg_hbm.at[sidx_v.at[win]], rows_v.at[pl.ds(win * 80, 80)], semg
            )
        )

    for q in range(BINS_W // L):
        utsm_v[0, pl.ds(q * L, L)] = (
            utsm_v[0, pl.ds(q * L, L)] + utsm_v[1, pl.ds(q * L, L)]
        )
    pltpu.sync_copy(utsm_v.at[0], uts_hbm.at[pl.ds(w * BINS_W, BINS_W)])

    def mbody(r16, carry):
        ff = maskf_v[pl.ds(r16 * L, L)]
        for li in range(L):
            fv = _take(ff, jnp.full((L,), li, jnp.int32))
            row = r16 * L + li
            for cc in range(D // L):
                sl = pl.ds(cc * L, L)
                rows_v[row, sl] = rows_v[row, sl] * fv
        return carry

    wcps = []
    for win in range(NWIN):
        gcps[win].wait()
        lax.fori_loop(win * 5, win * 5 + 5, mbody, 0)
        wcps.append(
            pltpu.async_copy(
                rows_v.at[pl.ds(win * 80, 80)],
                out_hbm.at[pl.ds(base + win * 80, 80)],
                semw,
            )
        )
    for cp in wcps:
        cp.wait()


def _sds(shape, dtype):
    return jax.ShapeDtypeStruct(shape, dtype)


_call_a = pl.kernel(
    _kern_a,
    out_type=(_sds((NW * M,), jnp.int32), _sds((N,), jnp.int32)),
    mesh=_mesh,
    compiler_params=pltpu.CompilerParams(needs_layout_passes=False),
    scratch_types=[
        pltpu.VMEM((CHUNK,), jnp.int32),
        pltpu.VMEM((M,), jnp.int32),
        pltpu.VMEM((CHUNK,), jnp.int32),
    ],
)

_call_b1 = pl.kernel(
    _kern_b1,
    out_type=(_sds((NW * M,), jnp.int32), _sds((M,), jnp.int32)),
    mesh=_mesh,
    compiler_params=pltpu.CompilerParams(needs_layout_passes=False),
    scratch_types=[
        pltpu.VMEM((NW, BINS_W), jnp.int32),
        pltpu.VMEM((NW, BINS_W), jnp.int32),
        pltpu.VMEM((BINS_W,), jnp.int32),
        pltpu.SemaphoreType.DMA,
    ],
)

_call_b2 = pl.kernel(
    _kern_b2,
    out_type=(_sds((2 * K * M,), jnp.int32), _sds((2 * M,), jnp.float32)),
    mesh=_mesh,
    compiler_params=pltpu.CompilerParams(needs_layout_passes=False),
    scratch_types=[
        pltpu.VMEM((CHUNK,), jnp.int32),
        pltpu.VMEM((CHUNK,), jnp.int32),
        pltpu.VMEM((CHUNK,), jnp.float32),
        pltpu.VMEM((M,), jnp.int32),
        pltpu.VMEM((M,), jnp.int32),
        pltpu.VMEM((CHUNK // 128, 128), jnp.int32),
        pltpu.VMEM((CHUNK // 128, 128), jnp.int32),
        pltpu.VMEM((CHUNK // 128, 128), jnp.int32),
        pltpu.VMEM((CHUNK // 128, 128), jnp.float32),
        pltpu.VMEM((SRC_SLC,), jnp.int32),
        pltpu.VMEM((TS_SLC,), jnp.float32),
        pltpu.VMEM_SHARED((TBL_S,), jnp.int32),
        pltpu.VMEM_SHARED((TBL_T,), jnp.float32),
        pltpu.SemaphoreType.DMA,
        pltpu.SemaphoreType.DMA,
        pltpu.SemaphoreType.DMA,
    ],
)

_call_c = pl.kernel(
    _kern_c,
    out_type=(_sds((K * M, D), jnp.float32), _sds((M,), jnp.float32)),
    mesh=_mesh,
    compiler_params=pltpu.CompilerParams(needs_layout_passes=False),
    scratch_types=[
        pltpu.VMEM((ROWS_W,), jnp.int32),
        pltpu.VMEM((ROWS_W,), jnp.int32),
        pltpu.VMEM((ROWS_W // 80, 80), jnp.int32),
        pltpu.VMEM((M,), jnp.int32),
        pltpu.VMEM((ROWS_W,), jnp.float32),
        pltpu.VMEM((2, BINS_W), jnp.float32),
        pltpu.VMEM((ROWS_W, D), jnp.float32),
        pltpu.SemaphoreType.DMA,
        pltpu.SemaphoreType.DMA,
        pltpu.SemaphoreType.DMA,
    ],
)


def kernel(messages, timestamps, node_ids):
    ids = node_ids.astype(jnp.int32)
    ts = timestamps.astype(jnp.float32)
    msgs = messages.astype(jnp.float32)
    hist, la = _call_a(ids)
    after, lengths = _call_b1(hist)
    src2, uts2 = _call_b2(ids, la, ts, after, lengths)
    um, uts = _call_c(msgs, src2, uts2, lengths)
    return (
        jnp.arange(M, dtype=node_ids.dtype),
        um.reshape(K, M, D),
        lengths.astype(node_ids.dtype),
        uts,
    )


# speedup vs baseline: 1.0182x; 1.0182x over previous
import jax
import jax.numpy as jnp
from jax import lax
from jax.experimental import pallas as pl
from jax.experimental.pallas import tpu as pltpu
from jax.experimental.pallas import tpu_sc as plsc

N, M, D = 32768, 2048, 256
K = 5
NC, NS, L = 2, 16, 16
NW = NC * NS
CHUNK = N // NW
VPC = CHUNK // L
BINS_W = M // NW
ROWS_W = (K * M) // NW
PAD = NW * L

_mesh = plsc.VectorSubcoreMesh(
    core_axis_name="c", subcore_axis_name="s", num_cores=NC, num_subcores=NS
)


def _wid():
    return lax.axis_index("s") * NC + lax.axis_index("c")


def _take(v, idx):
    return v.at[idx].get(mode="promise_in_bounds")


def _kern_a(ids_hbm, hist_hbm, la_hbm, ids_v, hist_v, la_v):
    w = _wid()
    base = w * CHUNK
    pltpu.sync_copy(ids_hbm.at[pl.ds(base, CHUNK)], ids_v)
    zero16 = jnp.zeros((L,), jnp.int32)

    def zbody(i, carry):
        hist_v[pl.ds(i * L, L)] = zero16
        return carry

    lax.fori_loop(0, M // L, zbody, 0)

    iota = lax.iota(jnp.int32, L)

    def body(tt, carry):
        t = VPC - 1 - tt
        v = ids_v[pl.ds(t * L, L)]
        later = jnp.zeros((L,), jnp.int32)
        tot = jnp.zeros((L,), jnp.int32)
        for s in range(1, L):
            rvec = _take(v, (iota + s) & (L - 1))
            eq = (rvec == v).astype(jnp.int32)
            tot = tot + eq
            later = later + eq * (iota < L - s).astype(jnp.int32)
        aft = plsc.load_gather(hist_v, [v])
        la_v[pl.ds(t * L, L)] = later + aft
        is_first = (tot - later) == 0
        plsc.addupdate_scatter(hist_v, [v], tot + 1, mask=is_first)
        return carry

    lax.fori_loop(0, VPC, body, 0)
    pltpu.sync_copy(hist_v, hist_hbm.at[pl.ds(w * M, M)])
    pltpu.sync_copy(la_v, la_hbm.at[pl.ds(base, CHUNK)])


def _kern_b1(hist_hbm, after_hbm, len_hbm, col_v, after_v, len_v, sem):
    w = _wid()
    b0 = w * BINS_W
    cps = [
        pltpu.async_copy(
            hist_hbm.at[pl.ds(wp * M + b0, BINS_W)], col_v.at[wp], sem
        )
        for wp in range(NW)
    ]
    for cp in cps:
        cp.wait()
    for q in range(BINS_W // L):
        acc = jnp.zeros((L,), jnp.int32)
        for wp in range(NW - 1, -1, -1):
            after_v[wp, pl.ds(q * L, L)] = acc
            acc = acc + col_v[wp, pl.ds(q * L, L)]
        len_v[pl.ds(q * L, L)] = jnp.minimum(acc, K)
    cps = [
        pltpu.async_copy(
            after_v.at[wp], after_hbm.at[pl.ds(wp * M + b0, BINS_W)], sem
        )
        for wp in range(NW)
    ]
    for cp in cps:
        cp.wait()
    pltpu.sync_copy(len_v, len_hbm.at[pl.ds(b0, BINS_W)])


TBL_S = K * M + NW * 128
TBL_T = M + NW * 128
SRC_SLC = K * M // NS
TS_SLC = M // NS


def _kern_b2(ids_hbm, la_hbm, ts_hbm, after_hbm, len_hbm, src2_out, uts2_out,
             ids_v, la_v, ts_v, after_v, len_v, didx_v, dts_v, vals_v,
             tsv_v, zsrc_v, zts_v, spm_src, spm_ts, semi, sem1, sem2):
    w = _wid()
    c = lax.axis_index("c")
    s = lax.axis_index("s")
    base = w * CHUNK
    zero16 = jnp.zeros((L,), jnp.int32)
    zero16f = jnp.zeros((L,), jnp.float32)

    def z1(i, carry):
        zsrc_v[pl.ds(i * L, L)] = zero16
        return carry

    lax.fori_loop(0, SRC_SLC // L, z1, 0)

    def z2(i, carry):
        zts_v[pl.ds(i * L, L)] = zero16f
        return carry

    lax.fori_loop(0, TS_SLC // L, z2, 0)
    pltpu.sync_copy(zsrc_v, spm_src.at[pl.ds(s * SRC_SLC, SRC_SLC)])
    pltpu.sync_copy(zts_v, spm_ts.at[pl.ds(s * TS_SLC, TS_SLC)])
    cps = [
        pltpu.async_copy(ids_hbm.at[pl.ds(base, CHUNK)], ids_v, semi),
        pltpu.async_copy(la_hbm.at[pl.ds(base, CHUNK)], la_v, semi),
        pltpu.async_copy(ts_hbm.at[pl.ds(base, CHUNK)], ts_v, semi),
        pltpu.async_copy(after_hbm.at[pl.ds(w * M, M)], after_v, semi),
        pltpu.async_copy(len_hbm, len_v, semi),
    ]
    for cp in cps:
        cp.wait()
    iota = lax.iota(jnp.int32, L)

    def body(j, carry):
        sl = pl.ds(j * L, L)
        m = ids_v[sl]
        la = la_v[sl]
        aft = plsc.load_gather(after_v, [m])
        pfe = la + aft
        ln = plsc.load_gather(len_v, [m])
        keep = pfe < K
        slot = ln - 1 - pfe
        dump = w * 128 + (j % 8) * L + iota
        d = jnp.where(keep, slot * M + m, K * M + dump)
        dts = jnp.where(pfe == 0, m, M + dump)
        row = j // 8
        col = (j % 8) * L
        didx_v[row, pl.ds(col, L)] = d
        dts_v[row, pl.ds(col, L)] = dts
        vals_v[row, pl.ds(col, L)] = base + j * L + iota + 1
        tsv_v[row, pl.ds(col, L)] = ts_v[sl]
        return carry

    lax.fori_loop(0, VPC, body, 0)
    plsc.subcore_barrier()
    cps = []
    for i in range(CHUNK // 128):
        cps.append(
            pltpu.async_copy(vals_v.at[i], spm_src.at[didx_v.at[i]], sem1, add=True)
        )
        cps.append(
            pltpu.async_copy(tsv_v.at[i], spm_ts.at[dts_v.at[i]], sem2, add=True)
        )
    for cp in cps:
        cp.wait()
    plsc.subcore_barrier()
    pltpu.sync_copy(
        spm_src.at[pl.ds(s * SRC_SLC, SRC_SLC)],
        src2_out.at[pl.ds(c * (K * M) + s * SRC_SLC, SRC_SLC)],
    )
    pltpu.sync_copy(
        spm_ts.at[pl.ds(s * TS_SLC, TS_SLC)],
        uts2_out.at[pl.ds(c * M + s * TS_SLC, TS_SLC)],
    )


def _kern_c(msg_hbm, src2_hbm, uts2_hbm, len_hbm, out_hbm, uts_hbm,
            sflat_v, sflatb_v, sidx_v, len_v, maskf_v, utsm_v, rows_v,
            semi, semg, semw):
    w = _wid()
    base = w * ROWS_W
    cps_in = [
        pltpu.async_copy(src2_hbm.at[pl.ds(base, ROWS_W)], sflat_v, semi),
        pltpu.async_copy(src2_hbm.at[pl.ds(K * M + base, ROWS_W)], sflatb_v, semi),
        pltpu.async_copy(len_hbm, len_v, semi),
        pltpu.async_copy(uts2_hbm.at[pl.ds(w * BINS_W, BINS_W)], utsm_v.at[0], semi),
        pltpu.async_copy(
            uts2_hbm.at[pl.ds(M + w * BINS_W, BINS_W)], utsm_v.at[1], semi
        ),
    ]
    for cp in cps_in:
        cp.wait()
    iota = lax.iota(jnp.int32, L)
    nmax = jnp.full((L,), N - 1, jnp.int32)
    zero = jnp.zeros((L,), jnp.int32)

    def cbody(q, carry):
        v = sflat_v[pl.ds(q * L, L)] + sflatb_v[pl.ds(q * L, L)] - 1
        v = jnp.minimum(jnp.maximum(v, zero), nmax)
        row = q // 5
        col = (q % 5) * L
        sidx_v[row, pl.ds(col, L)] = v
        d = base + q * L + iota
        m = d & (M - 1)
        s = jax.lax.shift_right_logical(d, 11)
        ln = plsc.load_gather(len_v, [m])
        maskf_v[pl.ds(q * L, L)] = (s < ln).astype(jnp.float32)
        return carry

    lax.fori_loop(0, ROWS_W // L, cbody, 0)

    for q in range(BINS_W // L):
        utsm_v[0, pl.ds(q * L, L)] = (
            utsm_v[0, pl.ds(q * L, L)] + utsm_v[1, pl.ds(q * L, L)]
        )
    pltpu.sync_copy(utsm_v.at[0], uts_hbm.at[pl.ds(w * BINS_W, BINS_W)])

    NWIN = ROWS_W // 80
    gcps = [
        pltpu.async_copy(
            msg_hbm.at[sidx_v.at[win]], rows_v.at[pl.ds(win * 80, 80)], semg
        )
        for win in range(NWIN)
    ]
    for cp in gcps:
        cp.wait()

    def mbody(r16, carry):
        ff = maskf_v[pl.ds(r16 * L, L)]
        for li in range(L):
            fv = _take(ff, jnp.full((L,), li, jnp.int32))
            row = r16 * L + li
            for cc in range(D // L):
                sl = pl.ds(cc * L, L)
                rows_v[row, sl] = rows_v[row, sl] * fv
        return carry

    lax.fori_loop(0, ROWS_W // L, mbody, 0)
    pltpu.sync_copy(rows_v, out_hbm.at[pl.ds(base, ROWS_W)])


def _sds(shape, dtype):
    return jax.ShapeDtypeStruct(shape, dtype)


_call_a = pl.kernel(
    _kern_a,
    out_type=(_sds((NW * M,), jnp.int32), _sds((N,), jnp.int32)),
    mesh=_mesh,
    compiler_params=pltpu.CompilerParams(needs_layout_passes=False),
    scratch_types=[
        pltpu.VMEM((CHUNK,), jnp.int32),
        pltpu.VMEM((M,), jnp.int32),
        pltpu.VMEM((CHUNK,), jnp.int32),
    ],
)

_call_b1 = pl.kernel(
    _kern_b1,
    out_type=(_sds((NW * M,), jnp.int32), _sds((M,), jnp.int32)),
    mesh=_mesh,
    compiler_params=pltpu.CompilerParams(needs_layout_passes=False),
    scratch_types=[
        pltpu.VMEM((NW, BINS_W), jnp.int32),
        pltpu.VMEM((NW, BINS_W), jnp.int32),
        pltpu.VMEM((BINS_W,), jnp.int32),
        pltpu.SemaphoreType.DMA,
    ],
)

_call_b2 = pl.kernel(
    _kern_b2,
    out_type=(_sds((2 * K * M,), jnp.int32), _sds((2 * M,), jnp.float32)),
    mesh=_mesh,
    compiler_params=pltpu.CompilerParams(needs_layout_passes=False),
    scratch_types=[
        pltpu.VMEM((CHUNK,), jnp.int32),
        pltpu.VMEM((CHUNK,), jnp.int32),
        pltpu.VMEM((CHUNK,), jnp.float32),
        pltpu.VMEM((M,), jnp.int32),
        pltpu.VMEM((M,), jnp.int32),
        pltpu.VMEM((CHUNK // 128, 128), jnp.int32),
        pltpu.VMEM((CHUNK // 128, 128), jnp.int32),
        pltpu.VMEM((CHUNK // 128, 128), jnp.int32),
        pltpu.VMEM((CHUNK // 128, 128), jnp.float32),
        pltpu.VMEM((SRC_SLC,), jnp.int32),
        pltpu.VMEM((TS_SLC,), jnp.float32),
        pltpu.VMEM_SHARED((TBL_S,), jnp.int32),
        pltpu.VMEM_SHARED((TBL_T,), jnp.float32),
        pltpu.SemaphoreType.DMA,
        pltpu.SemaphoreType.DMA,
        pltpu.SemaphoreType.DMA,
    ],
)

_call_c = pl.kernel(
    _kern_c,
    out_type=(_sds((K * M, D), jnp.float32), _sds((M,), jnp.float32)),
    mesh=_mesh,
    compiler_params=pltpu.CompilerParams(needs_layout_passes=False),
    scratch_types=[
        pltpu.VMEM((ROWS_W,), jnp.int32),
        pltpu.VMEM((ROWS_W,), jnp.int32),
        pltpu.VMEM((ROWS_W // 80, 80), jnp.int32),
        pltpu.VMEM((M,), jnp.int32),
        pltpu.VMEM((ROWS_W,), jnp.float32),
        pltpu.VMEM((2, BINS_W), jnp.float32),
        pltpu.VMEM((ROWS_W, D), jnp.float32),
        pltpu.SemaphoreType.DMA,
        pltpu.SemaphoreType.DMA,
        pltpu.SemaphoreType.DMA,
    ],
)


def kernel(messages, timestamps, node_ids):
    ids = node_ids.astype(jnp.int32)
    ts = timestamps.astype(jnp.float32)
    msgs = messages.astype(jnp.float32)
    hist, la = _call_a(ids)
    after, lengths = _call_b1(hist)
    src2, uts2 = _call_b2(ids, la, ts, after, lengths)
    um, uts = _call_c(msgs, src2, uts2, lengths)
    return (
        jnp.arange(M, dtype=node_ids.dtype),
        um.reshape(K, M, D),
        lengths.astype(node_ids.dtype),
        uts,
    )
